# Initial kernel scaffold; baseline (speedup 1.0000x reference)
import jax
import jax.numpy as jnp
from jax.experimental import pallas as pl


def kernel(x, edge_index, W_emb, b_emb, w_loc, b_loc, w_nb, b_nb):
    raise NotImplementedError("write your pallas kernel here")


# speedup vs baseline: 107.4605x; 107.4605x over previous
import functools

import jax
import jax.numpy as jnp
from jax import lax
from jax.experimental import pallas as pl
from jax.experimental.pallas import tpu as pltpu
from jax.experimental.pallas import tpu_sc as plsc

N = 10000
D = 128
E = 320000
NC = 2
NS = 16
NW = NC * NS
EPT = E // NW
L = 16


def _linears_body(x_ref, W_ref, be_ref, wcat_ref, bl_ref, bn_ref,
                  nb_ref, base_ref):
    hi = jax.lax.Precision.HIGHEST
    f32 = jnp.float32
    folded = lax.dot_general(W_ref[...], wcat_ref[...],
                             (((0,), (0,)), ((), ())),
                             precision=hi, preferred_element_type=f32)
    res = lax.dot_general(folded, x_ref[...],
                          (((0,), (1,)), ((), ())),
                          precision=hi, preferred_element_type=f32)
    bias = lax.dot_general(be_ref[...], wcat_ref[...],
                           (((1,), (0,)), ((), ())),
                           precision=hi, preferred_element_type=f32)
    bias_loc = bias[0:1, 0:1] + bl_ref[...]
    bias_nb = bias[0:1, 1:2] + bn_ref[...]
    nb_row = res[1:2, :] + bias_nb
    nb_ref[...] = nb_row
    base_ref[...] = res[0:1, :] + bias_loc + nb_row


def _scatter_body(edge_ref, nb_hbm, out_hbm, src_v, dst_v, nb_v, acc_v):
    wid = lax.axis_index("s") * NC + lax.axis_index("c")
    base = wid * EPT
    pltpu.sync_copy(edge_ref.at[pl.ds(base, EPT)], src_v)
    pltpu.sync_copy(edge_ref.at[pl.ds(E + base, EPT)], dst_v)
    pltpu.sync_copy(nb_hbm.at[0, :], nb_v)

    zeros = jnp.zeros((L,), jnp.float32)

    def zbody(i, carry):
        acc_v[pl.ds(i * L, L)] = zeros
        return carry

    lax.fori_loop(0, N // L, zbody, 0)

    def ebody(i, carry):
        s = src_v[pl.ds(i * L, L)]
        d = dst_v[pl.ds(i * L, L)]
        nbs = plsc.load_gather(nb_v, [s])
        nbd = plsc.load_gather(nb_v, [d])
        plsc.addupdate_scatter(acc_v, [s], nbd)
        plsc.addupdate_scatter(acc_v, [d], nbs)
        return carry

    lax.fori_loop(0, EPT // L, ebody, 0)
    pltpu.sync_copy(acc_v, out_hbm.at[wid])


def _reduce_body(part_ref, base_ref, out_ref):
    out_ref[...] = base_ref[...] + jnp.sum(part_ref[...], axis=0,
                                           keepdims=True)


def kernel(x, edge_index, W_emb, b_emb, w_loc, b_loc, w_nb, b_nb):
    f32 = jnp.float32
    wcat = jnp.concatenate([w_loc, w_nb], axis=1)
    be2 = b_emb.reshape(1, D)
    bl2 = b_loc.reshape(1, 1)
    bn2 = b_nb.reshape(1, 1)

    nb_row, base_row = pl.pallas_call(
        _linears_body,
        out_shape=[jax.ShapeDtypeStruct((1, N), f32),
                   jax.ShapeDtypeStruct((1, N), f32)],
    )(x, W_emb, be2, wcat, bl2, bn2)

    mesh = plsc.VectorSubcoreMesh(core_axis_name="c", subcore_axis_name="s",
                                  num_cores=NC, num_subcores=NS)
    partials = pl.kernel(
        _scatter_body,
        out_type=jax.ShapeDtypeStruct((NW, N), f32),
        mesh=mesh,
        scratch_types=[
            pltpu.VMEM((EPT,), jnp.int32),
            pltpu.VMEM((EPT,), jnp.int32),
            pltpu.VMEM((N,), f32),
            pltpu.VMEM((N,), f32),
        ],
        compiler_params=pltpu.CompilerParams(needs_layout_passes=False),
    )(edge_index.reshape(2 * E), nb_row)

    out_row = pl.pallas_call(
        _reduce_body,
        out_shape=jax.ShapeDtypeStruct((1, N), f32),
    )(partials, base_row)

    return out_row.reshape(N, 1)


# speedup vs baseline: 116.6509x; 1.0855x over previous
import functools

import jax
import jax.numpy as jnp
from jax import lax
from jax.experimental import pallas as pl
from jax.experimental.pallas import tpu as pltpu
from jax.experimental.pallas import tpu_sc as plsc

N = 10000
D = 128
E = 320000
NC = 2
NS = 16
NW = NC * NS
EPT = E // NW
L = 16


def _linears_body(x_ref, W_ref, be_ref, wcat_ref, bl_ref, bn_ref,
                  nb_ref, base_ref):
    hi = jax.lax.Precision.HIGHEST
    f32 = jnp.float32
    folded = lax.dot_general(W_ref[...], wcat_ref[...],
                             (((0,), (0,)), ((), ())),
                             precision=hi, preferred_element_type=f32)
    res = lax.dot_general(folded, x_ref[...],
                          (((0,), (1,)), ((), ())),
                          precision=hi, preferred_element_type=f32)
    bias = lax.dot_general(be_ref[...], wcat_ref[...],
                           (((1,), (0,)), ((), ())),
                           precision=hi, preferred_element_type=f32)
    bias_loc = bias[0:1, 0:1] + bl_ref[...]
    bias_nb = bias[0:1, 1:2] + bn_ref[...]
    nb_row = res[1:2, :] + bias_nb
    nb_ref[...] = nb_row
    base_ref[...] = res[0:1, :] + bias_loc + nb_row


def _scatter_body(edge_ref, nb_hbm, out_hbm, src_v, dst_v, nb_v, acc_v,
                  sem):
    wid = lax.axis_index("s") * NC + lax.axis_index("c")
    base = wid * EPT
    cp_s = pltpu.async_copy(edge_ref.at[pl.ds(base, EPT)], src_v, sem)
    cp_d = pltpu.async_copy(edge_ref.at[pl.ds(E + base, EPT)], dst_v, sem)
    cp_n = pltpu.async_copy(nb_hbm.at[0, :], nb_v, sem)

    zeros = jnp.zeros((L,), jnp.float32)

    def zbody(i, carry):
        acc_v[pl.ds(i * L, L)] = zeros
        return carry

    lax.fori_loop(0, N // L, zbody, 0, unroll=5)
    cp_s.wait()
    cp_d.wait()
    cp_n.wait()

    def ebody(i, carry):
        s = src_v[pl.ds(i * L, L)]
        d = dst_v[pl.ds(i * L, L)]
        nbs = plsc.load_gather(nb_v, [s])
        nbd = plsc.load_gather(nb_v, [d])
        plsc.addupdate_scatter(acc_v, [s], nbd)
        plsc.addupdate_scatter(acc_v, [d], nbs)
        return carry

    lax.fori_loop(0, EPT // L, ebody, 0, unroll=5)
    pltpu.sync_copy(acc_v, out_hbm.at[wid])


def _reduce_body(part_ref, base_ref, out_ref):
    out_ref[...] = base_ref[...] + jnp.sum(part_ref[...], axis=0,
                                           keepdims=True)


def kernel(x, edge_index, W_emb, b_emb, w_loc, b_loc, w_nb, b_nb):
    f32 = jnp.float32
    wcat = jnp.concatenate([w_loc, w_nb], axis=1)
    be2 = b_emb.reshape(1, D)
    bl2 = b_loc.reshape(1, 1)
    bn2 = b_nb.reshape(1, 1)

    nb_row, base_row = pl.pallas_call(
        _linears_body,
        out_shape=[jax.ShapeDtypeStruct((1, N), f32),
                   jax.ShapeDtypeStruct((1, N), f32)],
    )(x, W_emb, be2, wcat, bl2, bn2)

    mesh = plsc.VectorSubcoreMesh(core_axis_name="c", subcore_axis_name="s",
                                  num_cores=NC, num_subcores=NS)
    partials = pl.kernel(
        _scatter_body,
        out_type=jax.ShapeDtypeStruct((NW, N), f32),
        mesh=mesh,
        scratch_types=[
            pltpu.VMEM((EPT,), jnp.int32),
            pltpu.VMEM((EPT,), jnp.int32),
            pltpu.VMEM((N,), f32),
            pltpu.VMEM((N,), f32),
            pltpu.SemaphoreType.DMA,
        ],
        compiler_params=pltpu.CompilerParams(needs_layout_passes=False),
    )(edge_index.reshape(2 * E), nb_row)

    out_row = pl.pallas_call(
        _reduce_body,
        out_shape=jax.ShapeDtypeStruct((1, N), f32),
    )(partials, base_row)

    return out_row.reshape(N, 1)
